# baseline (speedup 1.0000x reference)
import jax
import jax.numpy as jnp
import numpy as np
from jax import lax
from jax.experimental import pallas as pl
from jax.experimental.pallas import tpu as pltpu
from jax.experimental.pallas import tpu_sc as plsc

E = 16
K = 2
N = 4096
D = 768
P = 4

BN = 512
NB = N // BN
BL = 1024
NBL = N // BL

NC = 2
NS = 16
NW = NC * NS
TPW = N // NW
NG = TPW // 16


def _logits_kernel(x_ref, gw_ref, pw_ref, bias_ref, out_ref, pwt_ref):
    out_ref[...] = lax.dot_general(
        gw_ref[...], x_ref[...], (((1,), (1,)), ((), ())),
        preferred_element_type=jnp.float32) + bias_ref[...]

    @pl.when(pl.program_id(0) == 0)
    def _():
        pwt_ref[...] = jnp.mean(pw_ref[...], axis=0).T


def _gate_sc_kernel(lt_hbm, pwt_hbm, wt_hbm, stats_hbm,
                    chunk_v, pwt_v, wbuf_v, stat_v, sem1, sem2):
    wid = lax.axis_index("s") * NC + lax.axis_index("c")
    base = wid * TPW
    cp1 = pltpu.async_copy(lt_hbm.at[:, pl.ds(base, TPW)], chunk_v, sem1)
    cp2 = pltpu.async_copy(pwt_hbm, pwt_v, sem2)
    cp1.wait()
    cp2.wait()

    neg = jnp.float32(-np.inf)
    pwt = [pwt_v[j, :] for j in range(E)]

    def _gather(v, idx):
        return v.at[idx].get(mode="promise_in_bounds")

    zero = jnp.zeros((16,), jnp.float32)
    sacc = [zero] * E
    hacc = [zero] * E

    for g in range(NG):
        sl = pl.ds(g * 16, 16)
        cols = [chunk_v[e, sl] for e in range(E)]
        m1 = cols[0]
        i1 = jnp.zeros((16,), jnp.int32)
        m2 = jnp.full((16,), neg)
        i2 = jnp.zeros((16,), jnp.int32)
        for e in range(1, E):
            v = cols[e]
            gt1 = v > m1
            gt2 = v > m2
            ev = jnp.full((16,), e, jnp.int32)
            i2 = jnp.where(gt1, i1, jnp.where(gt2, ev, i2))
            m2 = jnp.where(gt1, m1, jnp.where(gt2, v, m2))
            i1 = jnp.where(gt1, ev, i1)
            m1 = jnp.where(gt1, v, m1)
        mv1 = jnp.where(m1 == 0.0, neg, m1)
        mv2 = jnp.where(m2 == 0.0, neg, m2)
        mm = jnp.maximum(mv1, mv2)
        a1 = jnp.exp(mv1 - mm)
        a2 = jnp.exp(mv2 - mm)
        den = a1 + a2
        g1 = a1 / den
        g2 = a2 / den
        wj = []
        wsum = zero
        for j in range(E):
            w = g1 * _gather(pwt[j], i1) + g2 * _gather(pwt[j], i2)
            wj.append(w)
            wsum = wsum + w
        inv = 1.0 / wsum
        for j in range(E):
            wn = wj[j] * inv
            wbuf_v[j, sl] = wn
            sacc[j] = sacc[j] + wn
            hacc[j] = hacc[j] + jnp.where(wn >= 1e-5, 1.0, 0.0)

    cp3 = pltpu.async_copy(wbuf_v, wt_hbm.at[:, pl.ds(base, TPW)], sem1)
    for j in range(E):
        stat_v[j, :] = sacc[j]
        stat_v[E + j, :] = hacc[j]
    cp4 = pltpu.async_copy(stat_v, stats_hbm.at[wid], sem2)
    cp3.wait()
    cp4.wait()


def _combine_kernel(wt_ref, st_ref, f_ref, y_ref, soft_ref, hard_ref):
    i = pl.program_id(0)
    w = wt_ref[...].T
    acc = w[:, 0:1] * f_ref[0]
    for e in range(1, E):
        acc = acc + w[:, e:e + 1] * f_ref[e]
    y_ref[...] = acc

    @pl.when(i == 0)
    def _():
        st = jnp.sum(st_ref[...], axis=(0, 2)) / N
        soft_ref[...] = st[:E].reshape(1, E)
        hard_ref[...] = st[E:].reshape(1, E)


def kernel(f, x, permutation_weights, gate_weights, bias):
    bias_col = bias.reshape(E, 1)

    logits_t, pw_t = pl.pallas_call(
        _logits_kernel,
        grid=(NBL,),
        in_specs=[
            pl.BlockSpec((BL, D), lambda i: (i, 0)),
            pl.BlockSpec((E, D), lambda i: (0, 0)),
            pl.BlockSpec((P, E, E), lambda i: (0, 0, 0)),
            pl.BlockSpec((E, 1), lambda i: (0, 0)),
        ],
        out_specs=[
            pl.BlockSpec((E, BL), lambda i: (0, i)),
            pl.BlockSpec((E, E), lambda i: (0, 0)),
        ],
        out_shape=[
            jax.ShapeDtypeStruct((E, N), jnp.float32),
            jax.ShapeDtypeStruct((E, E), jnp.float32),
        ],
    )(x, gate_weights, permutation_weights, bias_col)

    gate_sc = pl.kernel(
        _gate_sc_kernel,
        mesh=plsc.VectorSubcoreMesh(core_axis_name="c", subcore_axis_name="s"),
        out_type=[
            jax.ShapeDtypeStruct((E, N), jnp.float32),
            jax.ShapeDtypeStruct((NW, 2 * E, 16), jnp.float32),
        ],
        scratch_types=[
            pltpu.VMEM((E, TPW), jnp.float32),
            pltpu.VMEM((E, E), jnp.float32),
            pltpu.VMEM((E, TPW), jnp.float32),
            pltpu.VMEM((2 * E, 16), jnp.float32),
            pltpu.SemaphoreType.DMA,
            pltpu.SemaphoreType.DMA,
        ],
    )
    wt, stat_parts = gate_sc(logits_t, pw_t)

    y, soft, hard = pl.pallas_call(
        _combine_kernel,
        grid=(NB,),
        in_specs=[
            pl.BlockSpec((E, BN), lambda i: (0, i)),
            pl.BlockSpec((NW, 2 * E, 16), lambda i: (0, 0, 0)),
            pl.BlockSpec((E, BN, D), lambda i: (0, i, 0)),
        ],
        out_specs=[
            pl.BlockSpec((BN, D), lambda i: (i, 0)),
            pl.BlockSpec((1, E), lambda i: (0, 0)),
            pl.BlockSpec((1, E), lambda i: (0, 0)),
        ],
        out_shape=[
            jax.ShapeDtypeStruct((N, D), jnp.float32),
            jax.ShapeDtypeStruct((1, E), jnp.float32),
            jax.ShapeDtypeStruct((1, E), jnp.float32),
        ],
    )(wt, stat_parts, f)
    return y, soft.reshape(E, 1), hard.reshape(E, 1)


# speedup vs baseline: 1.0217x; 1.0217x over previous
import jax
import jax.numpy as jnp
import numpy as np
from jax import lax
from jax.experimental import pallas as pl
from jax.experimental.pallas import tpu as pltpu
from jax.experimental.pallas import tpu_sc as plsc

E = 16
K = 2
N = 4096
D = 768
P = 4

BN = 256
NB = N // BN
BL = 2048
NBL = N // BL

NC = 2
NS = 16
NW = NC * NS
TPW = N // NW
NG = TPW // 16


def _logits_kernel(x_ref, gw_ref, pw_ref, bias_ref, out_ref, pwt_ref):
    out_ref[...] = lax.dot_general(
        gw_ref[...], x_ref[...], (((1,), (1,)), ((), ())),
        preferred_element_type=jnp.float32) + bias_ref[...]

    @pl.when(pl.program_id(0) == 0)
    def _():
        pwt_ref[...] = jnp.mean(pw_ref[...], axis=0).T


def _gate_sc_kernel(lt_hbm, pwt_hbm, wt_hbm, stats_hbm,
                    chunk_v, pwt_v, wbuf_v, stat_v, sem1, sem2):
    wid = lax.axis_index("s") * NC + lax.axis_index("c")
    base = wid * TPW
    cp1 = pltpu.async_copy(lt_hbm.at[:, pl.ds(base, TPW)], chunk_v, sem1)
    cp2 = pltpu.async_copy(pwt_hbm, pwt_v, sem2)
    cp1.wait()
    cp2.wait()

    neg = jnp.float32(-np.inf)
    pwt = [pwt_v[j, :] for j in range(E)]

    def _gather(v, idx):
        return v.at[idx].get(mode="promise_in_bounds")

    zero = jnp.zeros((16,), jnp.float32)
    sacc = [zero] * E
    hacc = [zero] * E

    for g in range(NG):
        sl = pl.ds(g * 16, 16)
        cols = [chunk_v[e, sl] for e in range(E)]
        m1 = cols[0]
        i1 = jnp.zeros((16,), jnp.int32)
        m2 = jnp.full((16,), neg)
        i2 = jnp.zeros((16,), jnp.int32)
        for e in range(1, E):
            v = cols[e]
            gt1 = v > m1
            gt2 = v > m2
            ev = jnp.full((16,), e, jnp.int32)
            i2 = jnp.where(gt1, i1, jnp.where(gt2, ev, i2))
            m2 = jnp.where(gt1, m1, jnp.where(gt2, v, m2))
            i1 = jnp.where(gt1, ev, i1)
            m1 = jnp.where(gt1, v, m1)
        mv1 = jnp.where(m1 == 0.0, neg, m1)
        mv2 = jnp.where(m2 == 0.0, neg, m2)
        mm = jnp.maximum(mv1, mv2)
        a1 = jnp.exp(mv1 - mm)
        a2 = jnp.exp(mv2 - mm)
        den = a1 + a2
        g1 = a1 / den
        g2 = a2 / den
        wj = []
        wsum = zero
        for j in range(E):
            w = g1 * _gather(pwt[j], i1) + g2 * _gather(pwt[j], i2)
            wj.append(w)
            wsum = wsum + w
        inv = 1.0 / wsum
        for j in range(E):
            wn = wj[j] * inv
            wbuf_v[j, sl] = wn
            sacc[j] = sacc[j] + wn
            hacc[j] = hacc[j] + jnp.where(wn >= 1e-5, 1.0, 0.0)

    cp3 = pltpu.async_copy(wbuf_v, wt_hbm.at[:, pl.ds(base, TPW)], sem1)
    for j in range(E):
        stat_v[j, :] = sacc[j]
        stat_v[E + j, :] = hacc[j]
    cp4 = pltpu.async_copy(stat_v, stats_hbm.at[wid], sem2)
    cp3.wait()
    cp4.wait()


def _combine_kernel(wt_ref, st_ref, f_ref, y_ref, soft_ref, hard_ref):
    i = pl.program_id(0)
    w = wt_ref[...].T
    acc = w[:, 0:1] * f_ref[0]
    for e in range(1, E):
        acc = acc + w[:, e:e + 1] * f_ref[e]
    y_ref[...] = acc

    @pl.when(i == 0)
    def _():
        st = jnp.sum(st_ref[...], axis=(0, 2)) / N
        soft_ref[...] = st[:E].reshape(1, E)
        hard_ref[...] = st[E:].reshape(1, E)


def kernel(f, x, permutation_weights, gate_weights, bias):
    bias_col = bias.reshape(E, 1)

    logits_t, pw_t = pl.pallas_call(
        _logits_kernel,
        grid=(NBL,),
        in_specs=[
            pl.BlockSpec((BL, D), lambda i: (i, 0)),
            pl.BlockSpec((E, D), lambda i: (0, 0)),
            pl.BlockSpec((P, E, E), lambda i: (0, 0, 0)),
            pl.BlockSpec((E, 1), lambda i: (0, 0)),
        ],
        out_specs=[
            pl.BlockSpec((E, BL), lambda i: (0, i)),
            pl.BlockSpec((E, E), lambda i: (0, 0)),
        ],
        out_shape=[
            jax.ShapeDtypeStruct((E, N), jnp.float32),
            jax.ShapeDtypeStruct((E, E), jnp.float32),
        ],
    )(x, gate_weights, permutation_weights, bias_col)

    gate_sc = pl.kernel(
        _gate_sc_kernel,
        mesh=plsc.VectorSubcoreMesh(core_axis_name="c", subcore_axis_name="s"),
        out_type=[
            jax.ShapeDtypeStruct((E, N), jnp.float32),
            jax.ShapeDtypeStruct((NW, 2 * E, 16), jnp.float32),
        ],
        scratch_types=[
            pltpu.VMEM((E, TPW), jnp.float32),
            pltpu.VMEM((E, E), jnp.float32),
            pltpu.VMEM((E, TPW), jnp.float32),
            pltpu.VMEM((2 * E, 16), jnp.float32),
            pltpu.SemaphoreType.DMA,
            pltpu.SemaphoreType.DMA,
        ],
    )
    wt, stat_parts = gate_sc(logits_t, pw_t)

    y, soft, hard = pl.pallas_call(
        _combine_kernel,
        grid=(NB,),
        in_specs=[
            pl.BlockSpec((E, BN), lambda i: (0, i)),
            pl.BlockSpec((NW, 2 * E, 16), lambda i: (0, 0, 0)),
            pl.BlockSpec((E, BN, D), lambda i: (0, i, 0)),
        ],
        out_specs=[
            pl.BlockSpec((BN, D), lambda i: (i, 0)),
            pl.BlockSpec((1, E), lambda i: (0, 0)),
            pl.BlockSpec((1, E), lambda i: (0, 0)),
        ],
        out_shape=[
            jax.ShapeDtypeStruct((N, D), jnp.float32),
            jax.ShapeDtypeStruct((1, E), jnp.float32),
            jax.ShapeDtypeStruct((1, E), jnp.float32),
        ],
    )(wt, stat_parts, f)
    return y, soft.reshape(E, 1), hard.reshape(E, 1)
